# baseline (speedup 1.0000x reference)
import functools

import jax
import jax.numpy as jnp
from jax import lax
from jax.experimental import pallas as pl
from jax.experimental.pallas import tpu as pltpu
from jax.experimental.pallas import tpu_sc as plsc

_NEG_INF = -1e30


def _lane_allsum(v, lane):
    for sh in (8, 4, 2, 1):
        perm = lax.bitwise_xor(lane, sh)
        v = v + v.at[perm].get(mode="promise_in_bounds")
    return v


def _sc_scores_kernel(B, C, D, V):
    info = plsc.get_sparse_core_info()
    NC, NS = info.num_cores, info.num_subcores
    NW = NC * NS
    CHUNK = B // NW
    S = 4
    NSUB = CHUNK // S
    NCTX = S * C
    G = D // 16
    inv_c = 1.0 / C

    mesh = plsc.VectorSubcoreMesh(core_axis_name="c", subcore_axis_name="s")

    @functools.partial(
        pl.kernel,
        out_type=jax.ShapeDtypeStruct((B * 16,), jnp.float32),
        mesh=mesh,
        scratch_types=[
            pltpu.VMEM((NCTX,), jnp.int32),
            pltpu.VMEM((NCTX * 8, D), jnp.float32),
            pltpu.VMEM((CHUNK + 16,), jnp.int32),
            pltpu.VMEM((S * 8, D), jnp.float32),
            pltpu.VMEM((16,), jnp.int32),
            pltpu.VMEM((8 * 8, D), jnp.float32),
            pltpu.VMEM((CHUNK * 16,), jnp.float32),
            pltpu.SemaphoreType.DMA,
        ],
    )
    def scores_kernel(x_hbm, pos_hbm, neg_hbm, w_hbm, out_hbm,
                      xidx_v, cblk_v, pidx_v, pblk_v, nidx_v, nblk_v,
                      sc_v, sem):
        cid = lax.axis_index("c")
        sid = lax.axis_index("s")
        wid = sid * NC + cid
        base = wid * CHUNK

        lane = lax.broadcasted_iota(jnp.int32, (16,), 0)

        def blk_start(scalar_idx):
            return pl.multiple_of(lax.bitwise_and(scalar_idx, ~7), 8)

        pltpu.sync_copy(neg_hbm, nidx_v)
        nall = nidx_v[...]
        nsub = []
        nh = []
        for k in range(5):
            nh.append(pltpu.async_copy(
                w_hbm.at[pl.ds(blk_start(nall[k]), 8)],
                nblk_v.at[pl.ds(k * 8, 8)], sem))
            nsub.append(lax.bitwise_and(nall[k], 7) + k * 8)
        for h in nh:
            h.wait()

        pltpu.sync_copy(pos_hbm.at[pl.ds(base, CHUNK)],
                        pidx_v.at[pl.ds(0, CHUNK)])

        def sub_chunk(j, _):
            pltpu.sync_copy(x_hbm.at[pl.ds(base * C + j * NCTX, NCTX)],
                            xidx_v)
            handles = []
            csub = []
            for q16 in range(NCTX // 16):
                iv = xidx_v[pl.ds(q16 * 16, 16)]
                for l in range(16):
                    q = q16 * 16 + l
                    e = iv[l]
                    handles.append(pltpu.async_copy(
                        w_hbm.at[pl.ds(blk_start(e), 8)],
                        cblk_v.at[pl.ds(q * 8, 8)], sem))
                    csub.append(lax.bitwise_and(e, 7) + q * 8)
            pv = pidx_v[pl.ds(j * S, 16)]
            psub = []
            for i in range(S):
                e = pv[i]
                handles.append(pltpu.async_copy(
                    w_hbm.at[pl.ds(blk_start(e), 8)],
                    pblk_v.at[pl.ds(i * 8, 8)], sem))
                psub.append(lax.bitwise_and(e, 7) + i * 8)
            for h in handles:
                h.wait()

            for i in range(S):
                rb = i * C
                acc = [cblk_v[csub[rb], pl.ds(g * 16, 16)]
                       for g in range(G)]
                for cc in range(1, C):
                    for g in range(G):
                        acc[g] = acc[g] + cblk_v[csub[rb + cc],
                                                 pl.ds(g * 16, 16)]

                ps = acc[0] * pblk_v[psub[i], pl.ds(0, 16)]
                for g in range(1, G):
                    ps = ps + acc[g] * pblk_v[psub[i], pl.ds(g * 16, 16)]
                s16 = jnp.full((16,), _NEG_INF, jnp.float32)
                s16 = jnp.where(lane == 0, _lane_allsum(ps, lane) * inv_c,
                                s16)

                for k in range(5):
                    ns = acc[0] * nblk_v[nsub[k], pl.ds(0, 16)]
                    for g in range(1, G):
                        ns = ns + acc[g] * nblk_v[nsub[k],
                                                  pl.ds(g * 16, 16)]
                    s16 = jnp.where(lane == k + 1,
                                    _lane_allsum(ns, lane) * inv_c, s16)

                sc_v[pl.ds((j * S + i) * 16, 16)] = s16
            return 0

        lax.fori_loop(0, NSUB, sub_chunk, 0)
        pltpu.sync_copy(sc_v, out_hbm.at[pl.ds(base * 16, CHUNK * 16)])

    return scores_kernel


def _tc_loss_kernel(scores_ref, out_ref):
    s = scores_ref[...]
    m = jnp.max(s, axis=1, keepdims=True)
    e = jnp.exp(s - m)
    lse = jnp.log(jnp.sum(e, axis=1)) + m[:, 0]
    out_ref[0, 0] = jnp.mean(lse - s[:, 0])


def kernel(x, pos_labels, neg_labels, W):
    B, C = x.shape
    V, D = W.shape
    x_flat = x.reshape(B * C)
    neg16 = jnp.concatenate(
        [neg_labels, jnp.zeros((16 - neg_labels.shape[0],), jnp.int32)])

    scores = _sc_scores_kernel(B, C, D, V)(x_flat, pos_labels, neg16, W)
    scores = scores.reshape(B, 16)

    loss = pl.pallas_call(
        _tc_loss_kernel,
        out_shape=jax.ShapeDtypeStruct((1, 1), jnp.float32),
        out_specs=pl.BlockSpec(memory_space=pltpu.SMEM),
    )(scores)
    return loss[0, 0]


# speedup vs baseline: 1.0394x; 1.0394x over previous
import functools

import jax
import jax.numpy as jnp
from jax import lax
from jax.experimental import pallas as pl
from jax.experimental.pallas import tpu as pltpu
from jax.experimental.pallas import tpu_sc as plsc

_NEG_INF = -1e30


def _lane_allsum(v, lane):
    for sh in (8, 4, 2, 1):
        perm = lax.bitwise_xor(lane, sh)
        v = v + v.at[perm].get(mode="promise_in_bounds")
    return v


def _sc_scores_kernel(B, C, D, V):
    info = plsc.get_sparse_core_info()
    NC, NS = info.num_cores, info.num_subcores
    NW = NC * NS
    CHUNK = B // NW
    S = 32
    NSUB = CHUNK // S
    IDXR = (S * C) // 128
    G = D // 16

    mesh = plsc.VectorSubcoreMesh(core_axis_name="c", subcore_axis_name="s")

    @functools.partial(
        pl.kernel,
        out_type=jax.ShapeDtypeStruct((B * 16,), jnp.float32),
        mesh=mesh,
        scratch_types=[
            pltpu.VMEM((S * C,), jnp.int32),
            pltpu.VMEM((S * C,), jnp.int32),
            pltpu.VMEM((S * C + 16,), jnp.int32),
            pltpu.VMEM((S * C, 128), jnp.float32),
            pltpu.VMEM((CHUNK,), jnp.int32),
            pltpu.VMEM((CHUNK,), jnp.int32),
            pltpu.VMEM((CHUNK + 16,), jnp.int32),
            pltpu.VMEM((S, 128), jnp.float32),
            pltpu.VMEM((16,), jnp.int32),
            pltpu.VMEM((16,), jnp.int32),
            pltpu.VMEM((16, 128), jnp.float32),
            pltpu.VMEM((CHUNK * 16,), jnp.float32),
            pltpu.SemaphoreType.DMA,
        ],
    )
    def scores_kernel(x_hbm, pos_hbm, neg_hbm, w_hbm, out_hbm,
                      xidx_v, xpk_v, xof_v, rows_v,
                      pidx_v, ppk_v, pof_v, pos_v,
                      nidx_v, npk_v, neg_v, sc_v, sem):
        cid = lax.axis_index("c")
        sid = lax.axis_index("s")
        wid = sid * NC + cid
        base = wid * CHUNK

        lane = lax.broadcasted_iota(jnp.int32, (16,), 0)

        pltpu.sync_copy(neg_hbm, nidx_v)
        npk_v[...] = lax.shift_right_logical(nidx_v[...], 1)
        pltpu.async_copy(w_hbm.at[npk_v], neg_v, sem).wait()

        pltpu.sync_copy(pos_hbm.at[pl.ds(base, CHUNK)], pidx_v)
        for q in range(CHUNK // 16):
            pv = pidx_v[pl.ds(q * 16, 16)]
            ppk_v[pl.ds(q * 16, 16)] = lax.shift_right_logical(pv, 1)
            pof_v[pl.ds(q * 16, 16)] = lax.shift_left(
                lax.bitwise_and(pv, 1), 6)

        def sub_chunk(j, _):
            pltpu.sync_copy(x_hbm.at[pl.ds(base * C + j * (S * C), S * C)],
                            xidx_v)
            for q in range((S * C) // 16):
                xv = xidx_v[pl.ds(q * 16, 16)]
                xpk_v[pl.ds(q * 16, 16)] = lax.shift_right_logical(xv, 1)
                xof_v[pl.ds(q * 16, 16)] = lax.shift_left(
                    lax.bitwise_and(xv, 1), 6)
            gs = [
                pltpu.async_copy(w_hbm.at[xpk_v.at[pl.ds(q * 128, 128)]],
                                 rows_v.at[pl.ds(q * 128, 128)], sem)
                for q in range(IDXR)
            ]
            gs.append(
                pltpu.async_copy(w_hbm.at[ppk_v.at[pl.ds(j * S, S)]],
                                 pos_v, sem))
            for g in gs:
                g.wait()

            nall = nidx_v[...]
            noffs = [lax.shift_left(lax.bitwise_and(nall[k], 1), 6)
                     for k in range(5)]

            def row_body(i, _):
                r = j * S + i
                rb = i * C
                ofa = xof_v[pl.ds(rb, 16)]
                ofb = xof_v[pl.ds(rb + 16, 16)]
                offs = [ofa[cc] for cc in range(16)]
                offs += [ofb[cc] for cc in range(C - 16)]
                acc = [rows_v[rb, pl.ds(offs[0] + g * 16, 16)]
                       for g in range(G)]
                for cc in range(1, C):
                    for g in range(G):
                        acc[g] = acc[g] + rows_v[rb + cc,
                                                 pl.ds(offs[cc] + g * 16, 16)]

                po = pof_v[pl.ds(r, 16)][0]
                ps = acc[0] * pos_v[i, pl.ds(po, 16)]
                for g in range(1, G):
                    ps = ps + acc[g] * pos_v[i, pl.ds(po + g * 16, 16)]
                s16 = jnp.full((16,), _NEG_INF, jnp.float32)
                s16 = jnp.where(lane == 0, _lane_allsum(ps, lane), s16)

                for k in range(5):
                    no = noffs[k]
                    ns = acc[0] * neg_v[k, pl.ds(no, 16)]
                    for g in range(1, G):
                        ns = ns + acc[g] * neg_v[k, pl.ds(no + g * 16, 16)]
                    s16 = jnp.where(lane == k + 1,
                                    _lane_allsum(ns, lane), s16)

                sc_v[pl.ds(r * 16, 16)] = s16
                return 0

            lax.fori_loop(0, S, row_body, 0)
            return 0

        lax.fori_loop(0, NSUB, sub_chunk, 0)
        pltpu.sync_copy(sc_v, out_hbm.at[pl.ds(base * 16, CHUNK * 16)])

    return scores_kernel


def _tc_loss_kernel(scores_ref, out_ref):
    s = scores_ref[...]
    m = jnp.max(s, axis=1, keepdims=True)
    e = jnp.exp(s - m)
    lse = jnp.log(jnp.sum(e, axis=1)) + m[:, 0]
    out_ref[0, 0] = jnp.mean(lse - s[:, 0])


def kernel(x, pos_labels, neg_labels, W):
    B, C = x.shape
    V, D = W.shape
    x_flat = x.reshape(B * C)
    neg16 = jnp.concatenate(
        [neg_labels, jnp.zeros((16 - neg_labels.shape[0],), jnp.int32)])
    w128 = (W * (float(C) ** -0.5)).reshape(V * D // 128, 128)

    scores = _sc_scores_kernel(B, C, D, V)(x_flat, pos_labels, neg16, w128)
    scores = scores.reshape(B, 16)

    loss = pl.pallas_call(
        _tc_loss_kernel,
        out_shape=jax.ShapeDtypeStruct((1, 1), jnp.float32),
        out_specs=pl.BlockSpec(memory_space=pltpu.SMEM),
    )(scores)
    return loss[0, 0]


# speedup vs baseline: 2.5578x; 2.4608x over previous
import functools

import jax
import jax.numpy as jnp
from jax import lax
from jax.experimental import pallas as pl
from jax.experimental.pallas import tpu as pltpu
from jax.experimental.pallas import tpu_sc as plsc

_NEG_INF = -1e30


def _lane_allsum(v, lane):
    for sh in (8, 4, 2, 1):
        perm = lax.bitwise_xor(lane, sh)
        v = v + v.at[perm].get(mode="promise_in_bounds")
    return v


def _tc_pack(W, C):
    V, D = W.shape
    WT = W.T
    BK = 32768
    grid = (V + BK - 1) // BK
    scale = float(C) ** -0.5

    def body(in_ref, out_ref):
        x = in_ref[...]
        lo = x[:, : BK // 2]
        hi = x[:, BK // 2:]
        z = jnp.concatenate([lo.T, hi.T], axis=1)
        out_ref[...] = z * scale

    return pl.pallas_call(
        body,
        grid=(grid,),
        in_specs=[pl.BlockSpec((D, BK), lambda i: (0, i))],
        out_specs=pl.BlockSpec((BK // 2, 2 * D), lambda i: (i, 0)),
        out_shape=jax.ShapeDtypeStruct((grid * BK // 2, 2 * D),
                                       jnp.float32),
    )(WT)


def _sc_scores_kernel(B, C, D, V):
    info = plsc.get_sparse_core_info()
    NC, NS = info.num_cores, info.num_subcores
    NW = NC * NS
    CHUNK = B // NW
    S = 16
    NSUB = CHUNK // S
    NCTX = S * C
    GSZ = 64
    IDXR = NCTX // GSZ
    G = D // 16

    mesh = plsc.VectorSubcoreMesh(core_axis_name="c", subcore_axis_name="s")

    def buf_set():
        return [
            pltpu.VMEM((NCTX,), jnp.int32),
            pltpu.VMEM((NCTX,), jnp.int32),
            pltpu.VMEM((NCTX + 16,), jnp.int32),
            pltpu.VMEM((NCTX, 2 * D), jnp.float32),
            pltpu.VMEM((S, 2 * D), jnp.float32),
            pltpu.SemaphoreType.DMA,
        ]

    @functools.partial(
        pl.kernel,
        out_type=jax.ShapeDtypeStruct((B * 16,), jnp.float32),
        mesh=mesh,
        scratch_types=buf_set() + buf_set() + [
            pltpu.VMEM((CHUNK,), jnp.int32),
            pltpu.VMEM((CHUNK,), jnp.int32),
            pltpu.VMEM((CHUNK + 16,), jnp.int32),
            pltpu.VMEM((16,), jnp.int32),
            pltpu.VMEM((16,), jnp.int32),
            pltpu.VMEM((16, 2 * D), jnp.float32),
            pltpu.VMEM((CHUNK * 16,), jnp.float32),
            pltpu.SemaphoreType.DMA,
        ],
    )
    def scores_kernel(x_hbm, pos_hbm, neg_hbm, w_hbm, out_hbm,
                      xidx_a, xpk_a, xof_a, rows_a, pos_a, sem_a,
                      xidx_b, xpk_b, xof_b, rows_b, pos_b, sem_b,
                      pidx_v, ppk_v, pof_v, nidx_v, npk_v, neg_v,
                      sc_v, sem_n):
        cid = lax.axis_index("c")
        sid = lax.axis_index("s")
        wid = sid * NC + cid
        base = wid * CHUNK

        lane = lax.broadcasted_iota(jnp.int32, (16,), 0)

        pltpu.sync_copy(neg_hbm, nidx_v)
        nv0 = nidx_v[...]
        npk_v[...] = lax.bitwise_or(
            lax.shift_left(lax.shift_right_logical(nv0, 15), 14),
            lax.bitwise_and(nv0, 16383))
        pltpu.async_copy(w_hbm.at[npk_v], neg_v, sem_n).wait()
        nhv = lax.shift_left(
            lax.bitwise_and(lax.shift_right_logical(nv0, 14), 1), 6)
        noffs = [nhv[k] for k in range(5)]

        pltpu.sync_copy(pos_hbm.at[pl.ds(base, CHUNK)],
                        pidx_v.at[pl.ds(0, CHUNK)])
        for q in range(CHUNK // 16):
            pv = pidx_v[pl.ds(q * 16, 16)]
            ppk_v[pl.ds(q * 16, 16)] = lax.bitwise_or(
                lax.shift_left(lax.shift_right_logical(pv, 15), 14),
                lax.bitwise_and(pv, 16383))
            pof_v[pl.ds(q * 16, 16)] = lax.shift_left(
                lax.bitwise_and(lax.shift_right_logical(pv, 14), 1), 6)

        def fire(jj, xidx_v, xpk_v, xof_v, rows_v, pos_v, sem):
            pltpu.sync_copy(x_hbm.at[pl.ds(base * C + jj * NCTX, NCTX)],
                            xidx_v)
            for q in range(NCTX // 16):
                xv = xidx_v[pl.ds(q * 16, 16)]
                xpk_v[pl.ds(q * 16, 16)] = lax.bitwise_or(
                    lax.shift_left(lax.shift_right_logical(xv, 15), 14),
                    lax.bitwise_and(xv, 16383))
                xof_v[pl.ds(q * 16, 16)] = lax.shift_left(
                    lax.bitwise_and(lax.shift_right_logical(xv, 14), 1), 6)
            for q in range(IDXR):
                pltpu.async_copy(
                    w_hbm.at[xpk_v.at[pl.ds(q * GSZ, GSZ)]],
                    rows_v.at[pl.ds(q * GSZ, GSZ)], sem)
            pltpu.async_copy(w_hbm.at[ppk_v.at[pl.ds(jj * S, S)]],
                             pos_v, sem)

        def drain(rows_v, pos_v, sem):
            for q in range(IDXR):
                pltpu.make_async_copy(
                    w_hbm.at[pl.ds(0, GSZ)],
                    rows_v.at[pl.ds(q * GSZ, GSZ)], sem).wait()
            pltpu.make_async_copy(w_hbm.at[pl.ds(0, S)], pos_v, sem).wait()

        def compute(jj, xof_v, rows_v, pos_v):
            def row_body(i, _):
                r = jj * S + i
                rb = i * C
                ofa = xof_v[pl.ds(rb, 16)]
                ofb = xof_v[pl.ds(rb + 16, 16)]
                offs = [ofa[cc] for cc in range(16)]
                offs += [ofb[cc] for cc in range(C - 16)]
                acc = [rows_v[rb, pl.ds(offs[0] + g * 16, 16)]
                       for g in range(G)]
                for cc in range(1, C):
                    for g in range(G):
                        acc[g] = acc[g] + rows_v[rb + cc,
                                                 pl.ds(offs[cc] + g * 16,
                                                       16)]

                po = pof_v[pl.ds(r, 16)][0]
                ps = acc[0] * pos_v[i, pl.ds(po, 16)]
                for g in range(1, G):
                    ps = ps + acc[g] * pos_v[i, pl.ds(po + g * 16, 16)]
                s16 = jnp.full((16,), _NEG_INF, jnp.float32)
                s16 = jnp.where(lane == 0, _lane_allsum(ps, lane), s16)

                for k in range(5):
                    no = noffs[k]
                    ns = acc[0] * neg_v[k, pl.ds(no, 16)]
                    for g in range(1, G):
                        ns = ns + acc[g] * neg_v[k, pl.ds(no + g * 16, 16)]
                    s16 = jnp.where(lane == k + 1,
                                    _lane_allsum(ns, lane), s16)

                sc_v[pl.ds(r * 16, 16)] = s16
                return 0

            lax.fori_loop(0, S, row_body, 0)

        fire(0, xidx_a, xpk_a, xof_a, rows_a, pos_a, sem_a)

        def pipelined(t, _):
            j0 = 2 * t
            j1 = 2 * t + 1
            j2 = jnp.where(j0 + 2 < NSUB, j0 + 2, NSUB - 2)
            drain(rows_a, pos_a, sem_a)
            fire(j1, xidx_b, xpk_b, xof_b, rows_b, pos_b, sem_b)
            compute(j0, xof_a, rows_a, pos_a)
            drain(rows_b, pos_b, sem_b)
            fire(j2, xidx_a, xpk_a, xof_a, rows_a, pos_a, sem_a)
            compute(j1, xof_b, rows_b, pos_b)
            return 0

        lax.fori_loop(0, NSUB // 2, pipelined, 0)
        drain(rows_a, pos_a, sem_a)

        pltpu.sync_copy(sc_v, out_hbm.at[pl.ds(base * 16, CHUNK * 16)])

    return scores_kernel


def _tc_loss_kernel(scores_ref, out_ref):
    s = scores_ref[...]
    m = jnp.max(s, axis=1, keepdims=True)
    e = jnp.exp(s - m)
    lse = jnp.log(jnp.sum(e, axis=1)) + m[:, 0]
    out_ref[0, 0] = jnp.mean(lse - s[:, 0])


def kernel(x, pos_labels, neg_labels, W):
    B, C = x.shape
    V, D = W.shape
    x_flat = x.reshape(B * C)
    neg16 = jnp.concatenate(
        [neg_labels, jnp.zeros((16 - neg_labels.shape[0],), jnp.int32)])

    w128 = _tc_pack(W, C)
    scores = _sc_scores_kernel(B, C, D, V)(x_flat, pos_labels, neg16, w128)
    scores = scores.reshape(B, 16)

    loss = pl.pallas_call(
        _tc_loss_kernel,
        out_shape=jax.ShapeDtypeStruct((1, 1), jnp.float32),
        out_specs=pl.BlockSpec(memory_space=pltpu.SMEM),
    )(scores)
    return loss[0, 0]
